# baseline (speedup 1.0000x reference)
import functools

import jax
import jax.numpy as jnp
from jax import lax
from jax.experimental import pallas as pl
from jax.experimental.pallas import tpu as pltpu
from jax.experimental.pallas import tpu_sc as plsc

B = 4096
T = 20
H = 256
D = 512
N = B * T

_info = plsc.get_sparse_core_info()
_NC = _info.num_cores
_NS = _info.num_subcores
_NW = _NC * _NS
_L = _info.num_lanes
_BPW = N // _NW
_CH = 80
_NPAIR = _BPW // (2 * _CH)


def _mlp_body(table_ref, w1_ref, b1_ref, w2_ref, b2_ref, out_ref):
    h = jnp.tanh(
        jnp.dot(table_ref[...], w1_ref[...], preferred_element_type=jnp.float32)
        + b1_ref[...]
    )
    out_ref[...] = (
        jnp.dot(h, w2_ref[...], preferred_element_type=jnp.float32) + b2_ref[...]
    )


def _transform_table(table, W1, b1, W2, b2):
    return pl.pallas_call(
        _mlp_body,
        out_shape=jax.ShapeDtypeStruct((T, D), jnp.float32),
    )(table, W1, b1.reshape(1, H), W2, b2.reshape(1, D))


_mesh = plsc.VectorSubcoreMesh(core_axis_name="c", subcore_axis_name="s")


@functools.partial(
    pl.kernel,
    mesh=_mesh,
    out_type=jax.ShapeDtypeStruct((N * D,), jnp.float32),
    scratch_types=[
        pltpu.VMEM((T * D,), jnp.float32),
        pltpu.VMEM((_BPW,), jnp.int32),
        pltpu.VMEM((_CH * D,), jnp.float32),
        pltpu.VMEM((_CH * D,), jnp.float32),
        pltpu.SemaphoreType.DMA,
        pltpu.SemaphoreType.DMA,
    ],
    compiler_params=pltpu.CompilerParams(needs_layout_passes=False),
)
def _sc_expand(tt_hbm, idx_hbm, out_hbm, tt_v, idx_v, buf0, buf1, sem0, sem1):
    wid = lax.axis_index("s") * _NC + lax.axis_index("c")
    base = wid * _BPW
    pltpu.sync_copy(tt_hbm, tt_v)
    pltpu.sync_copy(idx_hbm.at[pl.ds(base, _BPW)], idx_v)

    lane = lax.iota(jnp.int32, _L)
    dst_bases = [(g * _L + lane) * D for g in range(_CH // _L)]

    def compute(row_off, buf):
        for g in range(_CH // _L):
            idx16 = idx_v[pl.ds(row_off + g * _L, _L)]
            src_base = idx16 * D
            dst_base = dst_bases[g]

            def col(c, carry, src_base=src_base, dst_base=dst_base):
                v = plsc.load_gather(tt_v, [src_base + c])
                plsc.store_scatter(buf, [dst_base + c], v)
                return carry

            lax.fori_loop(0, D, col, 0, unroll=16)

    def scatter(row_off, buf, sem):
        return pltpu.async_copy(
            buf, out_hbm.at[pl.ds((base + row_off) * D, _CH * D)], sem
        )

    compute(0, buf0)

    def pair(p, carry):
        off0 = pl.multiple_of(p * 2 * _CH, 2 * _CH)
        cp0 = scatter(off0, buf0, sem0)
        compute(off0 + _CH, buf1)
        cp1 = scatter(off0 + _CH, buf1, sem1)
        cp0.wait()

        @pl.when(p < _NPAIR - 1)
        def _():
            compute(off0 + 2 * _CH, buf0)

        cp1.wait()
        return carry

    lax.fori_loop(0, _NPAIR, pair, 0)


def kernel(indices, table, W1, b1, W2, b2):
    ttable = _transform_table(table, W1, b1, W2, b2)
    out = _sc_expand(ttable.reshape(T * D), indices.reshape(N))
    return out.reshape(B, T, D)


# speedup vs baseline: 2.0374x; 2.0374x over previous
import functools

import jax
import jax.numpy as jnp
from jax import lax
from jax.experimental import pallas as pl
from jax.experimental.pallas import tpu as pltpu
from jax.experimental.pallas import tpu_sc as plsc

B = 4096
T = 20
H = 256
D = 512
N = B * T

_info = plsc.get_sparse_core_info()
_NC = _info.num_cores
_NS = _info.num_subcores
_NW = _NC * _NS
_L = _info.num_lanes
_BPW = N // _NW
_CH = 80
_NPAIR = _BPW // (2 * _CH)


def _mlp_body(table_ref, w1_ref, b1_ref, w2_ref, b2_ref, out_ref):
    h = jnp.tanh(
        jnp.dot(table_ref[...], w1_ref[...], preferred_element_type=jnp.float32)
        + b1_ref[...]
    )
    out_ref[...] = (
        jnp.dot(h, w2_ref[...], preferred_element_type=jnp.float32) + b2_ref[...]
    )


def _transform_table(table, W1, b1, W2, b2):
    return pl.pallas_call(
        _mlp_body,
        out_shape=jax.ShapeDtypeStruct((T, D), jnp.float32),
    )(table, W1, b1.reshape(1, H), W2, b2.reshape(1, D))


_mesh = plsc.VectorSubcoreMesh(core_axis_name="c", subcore_axis_name="s")


@functools.partial(
    pl.kernel,
    mesh=_mesh,
    out_type=jax.ShapeDtypeStruct((N * D,), jnp.float32),
    scratch_types=[
        pltpu.VMEM((T * D,), jnp.float32),
        pltpu.VMEM((_BPW,), jnp.int32),
        pltpu.VMEM((_CH * D,), jnp.float32),
        pltpu.VMEM((_CH * D,), jnp.float32),
        pltpu.SemaphoreType.DMA,
        pltpu.SemaphoreType.DMA,
    ],
    compiler_params=pltpu.CompilerParams(needs_layout_passes=False),
)
def _sc_expand(tt_hbm, idx_hbm, out_hbm, tt_v, idx_v, buf0, buf1, sem0, sem1):
    wid = lax.axis_index("s") * _NC + lax.axis_index("c")
    base = wid * _BPW
    pltpu.sync_copy(tt_hbm, tt_v)
    pltpu.sync_copy(idx_hbm.at[pl.ds(base, _BPW)], idx_v)

    lane = lax.iota(jnp.int32, _L)
    dst_bases = [(g * _L + lane) * D for g in range(_CH // _L)]

    def compute(row_off, buf):
        for g in range(_CH // _L):
            idx16 = idx_v[pl.ds(row_off + g * _L, _L)]
            src_base = idx16 * D
            dst_base = dst_bases[g]

            @plsc.parallel_loop(0, D, unroll=16)
            def col(c, src_base=src_base, dst_base=dst_base):
                v = plsc.load_gather(tt_v, [src_base + c])
                plsc.store_scatter(buf, [dst_base + c], v)

    def scatter(row_off, buf, sem):
        return pltpu.async_copy(
            buf, out_hbm.at[pl.ds((base + row_off) * D, _CH * D)], sem
        )

    compute(0, buf0)

    def pair(p, carry):
        off0 = pl.multiple_of(p * 2 * _CH, 2 * _CH)
        cp0 = scatter(off0, buf0, sem0)
        compute(off0 + _CH, buf1)
        cp1 = scatter(off0 + _CH, buf1, sem1)
        cp0.wait()

        @pl.when(p < _NPAIR - 1)
        def _():
            compute(off0 + 2 * _CH, buf0)

        cp1.wait()
        return carry

    lax.fori_loop(0, _NPAIR, pair, 0)


def kernel(indices, table, W1, b1, W2, b2):
    ttable = _transform_table(table, W1, b1, W2, b2)
    out = _sc_expand(ttable.reshape(T * D), indices.reshape(N))
    return out.reshape(B, T, D)


# speedup vs baseline: 2.4745x; 1.2145x over previous
import functools

import jax
import jax.numpy as jnp
from jax import lax
from jax.experimental import pallas as pl
from jax.experimental.pallas import tpu as pltpu
from jax.experimental.pallas import tpu_sc as plsc

B = 4096
T = 20
H = 256
D = 512
N = B * T

_info = plsc.get_sparse_core_info()
_NC = _info.num_cores
_NS = _info.num_subcores
_NW = _NC * _NS
_BPW = N // _NW
_CH = 80
_NPAIR = _BPW // (2 * _CH)


def _mlp_body(table_ref, w1_ref, b1_ref, w2_ref, b2_ref, out_ref):
    h = jnp.tanh(
        jnp.dot(table_ref[...], w1_ref[...], preferred_element_type=jnp.float32)
        + b1_ref[...]
    )
    out_ref[...] = (
        jnp.dot(h, w2_ref[...], preferred_element_type=jnp.float32) + b2_ref[...]
    )


def _transform_table(table, W1, b1, W2, b2):
    return pl.pallas_call(
        _mlp_body,
        out_shape=jax.ShapeDtypeStruct((T, D), jnp.float32),
    )(table, W1, b1.reshape(1, H), W2, b2.reshape(1, D))


_mesh = plsc.VectorSubcoreMesh(core_axis_name="c", subcore_axis_name="s")


@functools.partial(
    pl.kernel,
    mesh=_mesh,
    out_type=jax.ShapeDtypeStruct((N, D), jnp.float32),
    scratch_types=[
        pltpu.VMEM((_BPW,), jnp.int32),
        pltpu.VMEM((_CH, D), jnp.float32),
        pltpu.VMEM((_CH, D), jnp.float32),
        pltpu.SemaphoreType.DMA,
        pltpu.SemaphoreType.DMA,
    ],
)
def _sc_expand(tt_hbm, idx_hbm, out_hbm, idx_v, buf0, buf1, sem0, sem1):
    wid = lax.axis_index("s") * _NC + lax.axis_index("c")
    base = wid * _BPW
    pltpu.sync_copy(idx_hbm.at[pl.ds(base, _BPW)], idx_v)

    def gather(row_off, buf, sem):
        return pltpu.async_copy(
            tt_hbm.at[idx_v.at[pl.ds(row_off, _CH)]], buf, sem
        )

    def drain(row_off, buf, sem):
        return pltpu.async_copy(
            buf, out_hbm.at[pl.ds(base + row_off, _CH)], sem
        )

    gather(0, buf0, sem0).wait()

    def pair(p, carry):
        off0 = pl.multiple_of(p * 2 * _CH, 2 * _CH)
        cp0 = drain(off0, buf0, sem0)
        g1 = gather(off0 + _CH, buf1, sem1)
        g1.wait()
        cp1 = drain(off0 + _CH, buf1, sem1)
        cp0.wait()

        @pl.when(p < _NPAIR - 1)
        def _():
            gather(off0 + 2 * _CH, buf0, sem0).wait()

        cp1.wait()
        return carry

    lax.fori_loop(0, _NPAIR, pair, 0)


def kernel(indices, table, W1, b1, W2, b2):
    ttable = _transform_table(table, W1, b1, W2, b2)
    out = _sc_expand(ttable, indices.reshape(N))
    return out.reshape(B, T, D)


# speedup vs baseline: 3.8491x; 1.5555x over previous
import functools

import jax
import jax.numpy as jnp
from jax import lax
from jax.experimental import pallas as pl
from jax.experimental.pallas import tpu as pltpu
from jax.experimental.pallas import tpu_sc as plsc

B = 4096
T = 20
H = 256
D = 512
N = B * T

_info = plsc.get_sparse_core_info()
_NC = _info.num_cores
_NS = _info.num_subcores
_NW = _NC * _NS
_BPW = N // _NW
_CH = 80
_NPAIR = _BPW // (2 * _CH)


def _mlp_body(table_ref, w1_ref, b1_ref, w2_ref, b2_ref, out_ref):
    h = jnp.tanh(
        jnp.dot(table_ref[...], w1_ref[...], preferred_element_type=jnp.float32)
        + b1_ref[...]
    )
    y = jnp.dot(h, w2_ref[...], preferred_element_type=jnp.float32) + b2_ref[...]
    out_ref[...] = jnp.broadcast_to(y[None], (_NW, T, D))


def _transform_table(table, W1, b1, W2, b2):
    rep = pl.pallas_call(
        _mlp_body,
        out_shape=jax.ShapeDtypeStruct((_NW, T, D), jnp.float32),
    )(table, W1, b1.reshape(1, H), W2, b2.reshape(1, D))
    return rep.reshape(_NW * T, D)


_mesh = plsc.VectorSubcoreMesh(core_axis_name="c", subcore_axis_name="s")


@functools.partial(
    pl.kernel,
    mesh=_mesh,
    out_type=jax.ShapeDtypeStruct((N, D), jnp.float32),
    scratch_types=[
        pltpu.VMEM((_BPW,), jnp.int32),
        pltpu.VMEM((_CH, D), jnp.float32),
        pltpu.VMEM((_CH, D), jnp.float32),
        pltpu.SemaphoreType.DMA,
        pltpu.SemaphoreType.DMA,
    ],
)
def _sc_expand(tt_hbm, idx_hbm, out_hbm, idx_v, buf0, buf1, sem0, sem1):
    wid = lax.axis_index("s") * _NC + lax.axis_index("c")
    base = wid * _BPW
    pltpu.sync_copy(idx_hbm.at[pl.ds(base, _BPW)], idx_v)

    off = (wid * T).astype(jnp.int32)

    def bias(i, carry):
        p = pl.multiple_of(i * 16, 16)
        idx_v[pl.ds(p, 16)] = idx_v[pl.ds(p, 16)] + off
        return carry

    lax.fori_loop(0, _BPW // 16, bias, 0)

    def gather(row_off, buf, sem):
        return pltpu.async_copy(
            tt_hbm.at[idx_v.at[pl.ds(row_off, _CH)]], buf, sem
        )

    def drain(row_off, buf, sem):
        return pltpu.async_copy(
            buf, out_hbm.at[pl.ds(base + row_off, _CH)], sem
        )

    gather(0, buf0, sem0).wait()

    def pair(p, carry):
        off0 = pl.multiple_of(p * 2 * _CH, 2 * _CH)
        cp0 = drain(off0, buf0, sem0)
        g1 = gather(off0 + _CH, buf1, sem1)
        g1.wait()
        cp1 = drain(off0 + _CH, buf1, sem1)
        cp0.wait()

        @pl.when(p < _NPAIR - 1)
        def _():
            gather(off0 + 2 * _CH, buf0, sem0).wait()

        cp1.wait()
        return carry

    lax.fori_loop(0, _NPAIR, pair, 0)


def kernel(indices, table, W1, b1, W2, b2):
    ttable = _transform_table(table, W1, b1, W2, b2)
    out = _sc_expand(ttable, indices.reshape(N))
    return out.reshape(B, T, D)


# speedup vs baseline: 5.1765x; 1.3448x over previous
import functools

import jax
import jax.numpy as jnp
from jax import lax
from jax.experimental import pallas as pl
from jax.experimental.pallas import tpu as pltpu
from jax.experimental.pallas import tpu_sc as plsc

B = 4096
T = 20
H = 256
D = 512
N = B * T

_info = plsc.get_sparse_core_info()
_NC = _info.num_cores
_NS = _info.num_subcores
_NW = _NC * _NS
_BPW = N // _NW
_CH = 160
_NPAIR = _BPW // (2 * _CH)

_RB = 128
_HH = H // 2


def _mlp1_body(table_ref, w1_ref, b1_ref, outl_ref, outr_ref):
    h = jnp.tanh(
        jnp.dot(table_ref[...], w1_ref[...], preferred_element_type=jnp.float32)
        + b1_ref[...]
    )
    outl_ref[...] = jnp.broadcast_to(h[None, :, :_HH], (_NW, T, _HH))
    outr_ref[...] = jnp.broadcast_to(h[None, :, _HH:], (_NW, T, _HH))


def _activation_tables(table, W1, b1):
    hl, hr = pl.pallas_call(
        _mlp1_body,
        out_shape=[
            jax.ShapeDtypeStruct((_NW, T, _HH), jnp.float32),
            jax.ShapeDtypeStruct((_NW, T, _HH), jnp.float32),
        ],
    )(table, W1, b1.reshape(1, H))
    return hl.reshape(_NW * T, _HH), hr.reshape(_NW * T, _HH)


_mesh = plsc.VectorSubcoreMesh(core_axis_name="c", subcore_axis_name="s")


@functools.partial(
    pl.kernel,
    mesh=_mesh,
    out_type=[
        jax.ShapeDtypeStruct((N, _HH), jnp.float32),
        jax.ShapeDtypeStruct((N, _HH), jnp.float32),
    ],
    scratch_types=[
        pltpu.VMEM((_BPW,), jnp.int32),
        pltpu.VMEM((_CH, _HH), jnp.float32),
        pltpu.VMEM((_CH, _HH), jnp.float32),
        pltpu.VMEM((_CH, _HH), jnp.float32),
        pltpu.VMEM((_CH, _HH), jnp.float32),
        pltpu.SemaphoreType.DMA,
        pltpu.SemaphoreType.DMA,
        pltpu.SemaphoreType.DMA,
        pltpu.SemaphoreType.DMA,
    ],
)
def _sc_expand(
    hl_hbm, hr_hbm, idx_hbm, gl_hbm, gr_hbm,
    idx_v, bl0, bl1, br0, br1, sl0, sl1, sr0, sr1,
):
    wid = lax.axis_index("s") * _NC + lax.axis_index("c")
    base = wid * _BPW
    pltpu.sync_copy(idx_hbm.at[pl.ds(base, _BPW)], idx_v)

    off = (wid * T).astype(jnp.int32)

    def bias(i, carry):
        p = pl.multiple_of(i * 16, 16)
        idx_v[pl.ds(p, 16)] = idx_v[pl.ds(p, 16)] + off
        return carry

    lax.fori_loop(0, _BPW // 16, bias, 0)

    def gather(row_off, bufl, bufr, seml, semr):
        idx_slice = idx_v.at[pl.ds(row_off, _CH)]
        cl = pltpu.async_copy(hl_hbm.at[idx_slice], bufl, seml)
        cr = pltpu.async_copy(hr_hbm.at[idx_slice], bufr, semr)
        return cl, cr

    def drain(row_off, bufl, bufr, seml, semr):
        dst = pl.ds(base + row_off, _CH)
        cl = pltpu.async_copy(bufl, gl_hbm.at[dst], seml)
        cr = pltpu.async_copy(bufr, gr_hbm.at[dst], semr)
        return cl, cr

    def wait2(pair):
        pair[0].wait()
        pair[1].wait()

    wait2(gather(0, bl0, br0, sl0, sr0))

    def pair(p, carry):
        off0 = pl.multiple_of(p * 2 * _CH, 2 * _CH)
        d0 = drain(off0, bl0, br0, sl0, sr0)
        g1 = gather(off0 + _CH, bl1, br1, sl1, sr1)
        wait2(g1)
        d1 = drain(off0 + _CH, bl1, br1, sl1, sr1)
        wait2(d0)

        @pl.when(p < _NPAIR - 1)
        def _():
            wait2(gather(off0 + 2 * _CH, bl0, br0, sl0, sr0))

        wait2(d1)
        return carry

    lax.fori_loop(0, _NPAIR, pair, 0)


def _mlp2_body(gl_ref, gr_ref, w2a_ref, w2b_ref, b2_ref, out_ref):
    y = (
        jnp.dot(gl_ref[...], w2a_ref[...], preferred_element_type=jnp.float32)
        + jnp.dot(gr_ref[...], w2b_ref[...], preferred_element_type=jnp.float32)
        + b2_ref[...]
    )
    out_ref[...] = y.reshape(_RB, T, D)


def _dense_out(gl, gr, W2, b2):
    return pl.pallas_call(
        _mlp2_body,
        grid=(B // _RB,),
        in_specs=[
            pl.BlockSpec((_RB * T, _HH), lambda i: (i, 0)),
            pl.BlockSpec((_RB * T, _HH), lambda i: (i, 0)),
            pl.BlockSpec((_HH, D), lambda i: (0, 0)),
            pl.BlockSpec((_HH, D), lambda i: (0, 0)),
            pl.BlockSpec((1, D), lambda i: (0, 0)),
        ],
        out_specs=pl.BlockSpec((_RB, T, D), lambda i: (i, 0, 0)),
        out_shape=jax.ShapeDtypeStruct((B, T, D), jnp.float32),
    )(gl, gr, W2[:_HH], W2[_HH:], b2.reshape(1, D))


def kernel(indices, table, W1, b1, W2, b2):
    hl, hr = _activation_tables(table, W1, b1)
    gl, gr = _sc_expand(hl, hr, indices.reshape(N))
    return _dense_out(gl, gr, W2, b2)


# speedup vs baseline: 6.1282x; 1.1839x over previous
import functools

import jax
import jax.numpy as jnp
from jax import lax
from jax.experimental import pallas as pl
from jax.experimental.pallas import tpu as pltpu
from jax.experimental.pallas import tpu_sc as plsc

B = 4096
T = 20
H = 256
D = 512
N = B * T
K = 128

_info = plsc.get_sparse_core_info()
_NC = _info.num_cores
_NS = _info.num_subcores
_NW = _NC * _NS
_BPW = N // _NW
_CH = 320
_NPAIR = _BPW // (2 * _CH)

_RB = 128


def _table2_body(table_ref, w1_ref, b1_ref, w2_ref, b2_ref, out_ref):
    h = jnp.tanh(
        jnp.dot(table_ref[...], w1_ref[...], preferred_element_type=jnp.float32)
        + b1_ref[...]
    )
    t2 = jnp.dot(h, w2_ref[...], preferred_element_type=jnp.float32) + b2_ref[...]
    out_ref[...] = jnp.concatenate(
        [t2, jnp.zeros((K - T, D), jnp.float32)], axis=0
    )


def _padded_table2(table, W1, b1, W2, b2):
    return pl.pallas_call(
        _table2_body,
        out_shape=jax.ShapeDtypeStruct((K, D), jnp.float32),
    )(table, W1, b1.reshape(1, H), W2, b2.reshape(1, D))


_mesh = plsc.VectorSubcoreMesh(core_axis_name="c", subcore_axis_name="s")


@functools.partial(
    pl.kernel,
    mesh=_mesh,
    out_type=jax.ShapeDtypeStruct((N, K), jnp.float32),
    scratch_types=[
        pltpu.VMEM((_BPW,), jnp.int32),
        pltpu.VMEM((_CH, K), jnp.float32),
        pltpu.VMEM((_CH, K), jnp.float32),
        pltpu.SemaphoreType.DMA,
        pltpu.SemaphoreType.DMA,
    ],
)
def _sc_expand(oh_hbm, idx_hbm, out_hbm, idx_v, buf0, buf1, sem0, sem1):
    wid = lax.axis_index("s") * _NC + lax.axis_index("c")
    base = wid * _BPW
    pltpu.sync_copy(idx_hbm.at[pl.ds(base, _BPW)], idx_v)

    off = (wid * T).astype(jnp.int32)

    def bias(i, carry):
        p = pl.multiple_of(i * 16, 16)
        idx_v[pl.ds(p, 16)] = idx_v[pl.ds(p, 16)] + off
        return carry

    lax.fori_loop(0, _BPW // 16, bias, 0)

    def gather(row_off, buf, sem):
        return pltpu.async_copy(
            oh_hbm.at[idx_v.at[pl.ds(row_off, _CH)]], buf, sem
        )

    def drain(row_off, buf, sem):
        return pltpu.async_copy(
            buf, out_hbm.at[pl.ds(base + row_off, _CH)], sem
        )

    gather(0, buf0, sem0).wait()

    def pair(p, carry):
        off0 = pl.multiple_of(p * 2 * _CH, 2 * _CH)
        cp0 = drain(off0, buf0, sem0)
        g1 = gather(off0 + _CH, buf1, sem1)
        g1.wait()
        cp1 = drain(off0 + _CH, buf1, sem1)
        cp0.wait()

        @pl.when(p < _NPAIR - 1)
        def _():
            gather(off0 + 2 * _CH, buf0, sem0).wait()

        cp1.wait()
        return carry

    lax.fori_loop(0, _NPAIR, pair, 0)


def _select_body(oh_ref, t2_ref, out_ref):
    y = jnp.dot(oh_ref[...], t2_ref[...], preferred_element_type=jnp.float32)
    out_ref[...] = y.reshape(_RB, T, D)


def _dense_out(oh, t2p):
    return pl.pallas_call(
        _select_body,
        grid=(B // _RB,),
        in_specs=[
            pl.BlockSpec((_RB * T, K), lambda i: (i, 0)),
            pl.BlockSpec((K, D), lambda i: (0, 0)),
        ],
        out_specs=pl.BlockSpec((_RB, T, D), lambda i: (i, 0, 0)),
        out_shape=jax.ShapeDtypeStruct((B, T, D), jnp.float32),
    )(oh, t2p)


def kernel(indices, table, W1, b1, W2, b2):
    t2p = _padded_table2(table, W1, b1, W2, b2)
    oh_table = jnp.tile(jnp.eye(T, K, dtype=jnp.float32), (_NW, 1))
    oh = _sc_expand(oh_table, indices.reshape(N))
    return _dense_out(oh, t2p)
